# baseline (speedup 1.0000x reference)
import functools

import jax
import jax.numpy as jnp
from jax import lax
from jax.experimental import pallas as pl
from jax.experimental.pallas import tpu as pltpu
from jax.experimental.pallas import tpu_sc as plsc

N = 10000
D = 128
H = 128
G = 64
OUT = 3

NP = 10240
BM = 512
NB = NP // BM
NC = 2
NS = 16
NW = NC * NS
EB = 112
GP = 72
RPT = NP // NS
NT = NP
ZC = 64
SLOW_FRAC = 0.50

_f32 = jnp.float32
_i32 = jnp.int32


def _sc_scalar_kernel(ept,
                      asrc_hbm, adst_hbm, src_hbm, dst_hbm,
                      ex_out, s_out,
                      asrc_v, adst_v, s_acc, src_v, dst_v, ex_v):
    cid = lax.axis_index("c")
    sid = lax.axis_index("s")
    wid = cid * NS + sid

    zero16 = jnp.zeros((16,), _f32)

    def _zs(j, _):
        s_acc[pl.ds(j * 16, 16)] = zero16
        return 0

    lax.fori_loop(0, NT // 16, _zs, 0)

    pltpu.sync_copy(asrc_hbm.at[pl.ds(0, NT)], asrc_v)
    pltpu.sync_copy(adst_hbm.at[pl.ds(0, NT)], adst_v)
    pltpu.sync_copy(src_hbm.at[pl.ds(wid * ept, ept)], src_v)
    pltpu.sync_copy(dst_hbm.at[pl.ds(wid * ept, ept)], dst_v)

    iota16 = lax.iota(_i32, 16)

    def _chunk(c, _):
        s16 = src_v[pl.ds(c * 16, 16)]
        d16 = dst_v[pl.ds(c * 16, 16)]
        av = plsc.load_gather(asrc_v, [s16])
        dv = plsc.load_gather(adst_v, [d16])
        e = av + dv
        e = jnp.where(e >= 0.0, e, 0.2 * e)
        ex = jnp.exp(e)
        ex_v[pl.ds(c * 16, 16)] = ex
        for l in range(16):
            plsc.addupdate_scatter(s_acc, [d16], ex, mask=iota16 == l)
        return 0

    lax.fori_loop(0, ept // 16, _chunk, 0)

    pltpu.sync_copy(ex_v, ex_out.at[pl.ds(wid * ept, ept)])
    pltpu.sync_copy(s_acc, s_out.at[wid, 0, pl.ds(0, NT)])


def _sc_scalar_pass(asrc, adst, src, dst, ept, ep):
    body = functools.partial(_sc_scalar_kernel, ept)
    ex, sparts = pl.kernel(
        body,
        out_type=[
            jax.ShapeDtypeStruct((ep,), _f32),
            jax.ShapeDtypeStruct((NW, 1, NP), _f32),
        ],
        mesh=plsc.VectorSubcoreMesh(core_axis_name="c", subcore_axis_name="s",
                                    num_cores=NC, num_subcores=NS),
        compiler_params=pltpu.CompilerParams(needs_layout_passes=False),
        scratch_types=[
            pltpu.VMEM((NT,), _f32),
            pltpu.VMEM((NT,), _f32),
            pltpu.VMEM((NT,), _f32),
            pltpu.VMEM((ept,), _i32),
            pltpu.VMEM((ept,), _i32),
            pltpu.VMEM((ept,), _f32),
        ],
    )(asrc, adst, src, dst)
    return ex, sparts.reshape(NW, NP)


def _sc_heavy_kernel(epb0, epb1,
                     h_hbm, src_hbm, dst_hbm, ex_hbm,
                     usum_out,
                     usum_sh,
                     src0, src1, dst0, dst1, exb0, exb1, dsc0, dsc1,
                     rows0, rows1, sb0, sb1,
                     isem0, isem1, gsem0, gsem1, ssem0, ssem1):
    cid = lax.axis_index("c")
    sid = lax.axis_index("s")
    wid = cid * NS + sid

    srcs = (src0, src1)
    dsts = (dst0, dst1)
    exbs = (exb0, exb1)
    dscs = (dsc0, dsc1)
    rows = (rows0, rows1)
    sbs = (sb0, sb1)
    isems = (isem0, isem1)
    gsems = (gsem0, gsem1)
    ssems = (ssem0, ssem1)

    zero16 = jnp.zeros((16,), _f32)

    def _zrow(j, _):
        for k in range(8):
            sb0[j, pl.ds(k * 16, 16)] = zero16
        return 0

    lax.fori_loop(0, EB, _zrow, 0)

    for i in range(RPT // ZC):
        base = sid * RPT + i * ZC
        pltpu.sync_copy(sb0.at[pl.ds(0, ZC), :],
                        usum_sh.at[pl.ds(base, ZC), :])

    plsc.subcore_barrier()

    epbc = jnp.where(cid == 0, epb0, epb1)
    ebase = jnp.where(cid == 0, sid * epb0, NS * epb0 + sid * epb1) * EB

    def _idx_start(blk, p):
        off = ebase + blk * EB
        pltpu.make_async_copy(src_hbm.at[pl.ds(off, EB)], srcs[p],
                              isems[p]).start()
        pltpu.make_async_copy(dst_hbm.at[pl.ds(off, EB)], dsts[p],
                              isems[p]).start()
        pltpu.make_async_copy(ex_hbm.at[pl.ds(off, EB)], exbs[p],
                              isems[p]).start()

    def _idx_wait(blk, p):
        off = ebase + blk * EB
        pltpu.make_async_copy(src_hbm.at[pl.ds(off, EB)], srcs[p],
                              isems[p]).wait()
        pltpu.make_async_copy(dst_hbm.at[pl.ds(off, EB)], dsts[p],
                              isems[p]).wait()
        pltpu.make_async_copy(ex_hbm.at[pl.ds(off, EB)], exbs[p],
                              isems[p]).wait()

    def _gather_start(p):
        pltpu.make_async_copy(h_hbm.at[srcs[p]], rows[p], gsems[p]).start()

    def _gather_wait(p):
        pltpu.make_async_copy(h_hbm.at[srcs[p]], rows[p], gsems[p]).wait()

    def _scat_start(p):
        pltpu.async_copy(sbs[p], usum_sh.at[dscs[p]], ssems[p], add=True)

    def _scat_wait(p):
        pltpu.make_async_copy(sbs[p], usum_sh.at[dscs[p]], ssems[p]).wait()

    _idx_start(0, 0)
    _idx_start(1, 1)
    _idx_wait(0, 0)
    _gather_start(0)

    def _step(blk, p, first=False):
        _idx_wait(blk + 1, 1 - p)
        _gather_start(1 - p)
        _gather_wait(p)

        dst_v, ex_v, dsc_v, rows_v, sb_v = (dsts[p], exbs[p], dscs[p],
                                            rows[p], sbs[p])
        for k in range(EB // 16):
            dsc_v[pl.ds(k * 16, 16)] = dst_v[pl.ds(k * 16, 16)]

        if not first:
            _scat_wait(p)

        def _scale(j, _):
            b = plsc.load_gather(ex_v, [jnp.full((16,), j, _i32)])
            for k in range(4):
                v = rows_v[j, pl.ds(k * 16, 16)]
                lo, hi = plsc.unpack(plsc.bitcast(v, jnp.bfloat16),
                                     format=plsc.PackFormat.INTERLEAVED)
                sb_v[j, pl.ds(k * 16, 16)] = lo * b
                sb_v[j, pl.ds(64 + k * 16, 16)] = hi * b
            return 0

        lax.fori_loop(0, EB, _scale, 0)

        _scat_start(p)
        _idx_start(blk + 2, p)

    _step(0, 0, first=True)
    _step(1, 1, first=True)

    def _pair(o, _):
        _step(2 * o + 2, 0)
        _step(2 * o + 3, 1)
        return 0

    lax.fori_loop(0, (epbc - 2) // 2, _pair, 0)

    _idx_wait(epbc + 1, 1)
    _gather_wait(0)
    _scat_wait(0)
    _scat_wait(1)

    plsc.subcore_barrier()

    pltpu.sync_copy(usum_sh.at[pl.ds(sid * RPT, RPT), :],
                    usum_out.at[cid, pl.ds(sid * RPT, RPT), :])


def _sc_heavy_pass(h, src, dst, ex, epb0, epb1):
    body = functools.partial(_sc_heavy_kernel, epb0, epb1)
    return pl.kernel(
        body,
        out_type=jax.ShapeDtypeStruct((NC, NP, H), _f32),
        mesh=plsc.VectorSubcoreMesh(core_axis_name="c", subcore_axis_name="s",
                                    num_cores=NC, num_subcores=NS),
        compiler_params=pltpu.CompilerParams(needs_layout_passes=False,
                                             use_tc_tiling_on_sc=False),
        scratch_types=[
            pltpu.VMEM_SHARED((NP, H), _f32),
            pltpu.VMEM((EB,), _i32),
            pltpu.VMEM((EB,), _i32),
            pltpu.VMEM((EB,), _i32),
            pltpu.VMEM((EB,), _i32),
            pltpu.VMEM((EB,), _f32),
            pltpu.VMEM((EB,), _f32),
            pltpu.VMEM((EB,), _i32),
            pltpu.VMEM((EB,), _i32),
            pltpu.VMEM((EB, H // 2), _f32),
            pltpu.VMEM((EB, H // 2), _f32),
            pltpu.VMEM((EB, H), _f32),
            pltpu.VMEM((EB, H), _f32),
            pltpu.SemaphoreType.DMA,
            pltpu.SemaphoreType.DMA,
            pltpu.SemaphoreType.DMA,
            pltpu.SemaphoreType.DMA,
            pltpu.SemaphoreType.DMA,
            pltpu.SemaphoreType.DMA,
        ],
    )(h, src, dst, ex)


def _pack_rows(hv):
    lob = lax.bitcast_convert_type(hv[:, :64].astype(jnp.bfloat16),
                                   jnp.uint16).astype(jnp.uint32)
    hib = lax.bitcast_convert_type(hv[:, 64:].astype(jnp.bfloat16),
                                   jnp.uint16).astype(jnp.uint32)
    return lax.bitcast_convert_type(lob | (hib << 16), _f32)


def _tc_embed_body(x_ref, w_ref, aa_ref, h_ref, a_ref):
    xr = jnp.maximum(x_ref[...], 0.0)
    hv = jnp.dot(xr, w_ref[...], preferred_element_type=_f32)
    h_ref[...] = _pack_rows(hv)
    a_ref[...] = jnp.dot(hv, aa_ref[...], preferred_element_type=_f32)


def _tc_embed(x_pad, w, aa):
    return pl.pallas_call(
        _tc_embed_body,
        grid=(NB,),
        in_specs=[
            pl.BlockSpec((BM, D), lambda i: (i, 0)),
            pl.BlockSpec((D, H), lambda i: (0, 0)),
            pl.BlockSpec((H, 16), lambda i: (0, 0)),
        ],
        out_specs=[
            pl.BlockSpec((BM, H // 2), lambda i: (i, 0)),
            pl.BlockSpec((BM, 16), lambda i: (i, 0)),
        ],
        out_shape=[
            jax.ShapeDtypeStruct((NP, H // 2), _f32),
            jax.ShapeDtypeStruct((NP, 16), _f32),
        ],
    )(x_pad, w, aa)


def _den_from_parts(sp):
    ones = jnp.ones((NW, 8), _f32)
    tot = lax.dot_general(sp, ones, (((0,), (0,)), ((), ())),
                          preferred_element_type=_f32)
    return jnp.maximum(tot[:, 0:1], 1e-16)


def _tc_mid_body(u0_ref, u1_ref, sp_ref, b_ref, w_ref, aa_ref,
                 h_ref, a_ref):
    i = pl.program_id(0)
    u = u0_ref[...] + u1_ref[...]
    den = _den_from_parts(sp_ref[...])
    h1 = u / den + b_ref[...]
    x2 = jnp.maximum(h1, 0.0)
    ridx = lax.broadcasted_iota(_i32, (BM, 1), 0) + i * BM
    x2 = jnp.where(ridx < N, x2, 0.0)
    hv = jnp.dot(x2, w_ref[...], preferred_element_type=_f32)
    h_ref[...] = _pack_rows(hv)
    a_ref[...] = jnp.dot(hv, aa_ref[...], preferred_element_type=_f32)


def _tc_mid(usum, sparts, b, w, aa):
    return pl.pallas_call(
        _tc_mid_body,
        grid=(NB,),
        in_specs=[
            pl.BlockSpec((BM, H), lambda i: (i, 0)),
            pl.BlockSpec((BM, H), lambda i: (i, 0)),
            pl.BlockSpec((NW, BM), lambda i: (0, i)),
            pl.BlockSpec((1, H), lambda i: (0, 0)),
            pl.BlockSpec((H, H), lambda i: (0, 0)),
            pl.BlockSpec((H, 16), lambda i: (0, 0)),
        ],
        out_specs=[
            pl.BlockSpec((BM, H // 2), lambda i: (i, 0)),
            pl.BlockSpec((BM, 16), lambda i: (i, 0)),
        ],
        out_shape=[
            jax.ShapeDtypeStruct((NP, H // 2), _f32),
            jax.ShapeDtypeStruct((NP, 16), _f32),
        ],
    )(usum[0], usum[1], sparts, b, w, aa)


def _tc_pool_body(u0_ref, u1_ref, sp_ref, b_ref, bat_ref,
                  wf_ref, bf_ref, out_ref, gsum, gmax, gcnt):
    i = pl.program_id(0)

    @pl.when(i == 0)
    def _():
        gsum[...] = jnp.zeros((GP, H), _f32)
        gcnt[...] = jnp.zeros((GP, H), _f32)
        gmax[...] = jnp.full((GP, H), -jnp.inf, _f32)

    u = u0_ref[...] + u1_ref[...]
    den = _den_from_parts(sp_ref[...])
    h2 = u / den + b_ref[...]
    ridx = lax.broadcasted_iota(_i32, (BM, 1), 0) + i * BM
    h2 = jnp.where(ridx < N, h2, 0.0)

    bcol = bat_ref[...]
    giota = lax.broadcasted_iota(_i32, (BM, GP), 1)
    oh = (bcol == giota).astype(_f32)
    dn = (((0,), (0,)), ((), ()))
    gsum[...] = gsum[...] + lax.dot_general(oh, h2, dn,
                                            preferred_element_type=_f32)
    gcnt[...] = gcnt[...] + lax.dot_general(oh, jnp.ones_like(h2), dn,
                                            preferred_element_type=_f32)

    b_lo = jnp.min(bcol)
    b_hi = jnp.max(bcol)

    def _gmax(g, _):
        @pl.when(jnp.logical_and(g >= b_lo, g <= b_hi))
        def _():
            m = bcol == g
            mx = jnp.max(jnp.where(m, h2, -jnp.inf), axis=0, keepdims=True)
            gmax[pl.ds(g, 1), :] = jnp.maximum(gmax[pl.ds(g, 1), :], mx)
        return 0

    lax.fori_loop(0, G, _gmax, 0)

    @pl.when(i == NB - 1)
    def _():
        cnt = gcnt[...]
        gmx = jnp.where(cnt > 0.0, gmax[...], 0.0)
        gmn = gsum[...] / jnp.maximum(cnt, 1.0)
        pooled = jnp.concatenate([gmx[:G], gmn[:G]], axis=1)
        out_ref[...] = (jnp.dot(pooled, wf_ref[...],
                                preferred_element_type=_f32) + bf_ref[...])


def _tc_pool(usum, sparts, b, bat2d, wfp, bfp):
    return pl.pallas_call(
        _tc_pool_body,
        grid=(NB,),
        in_specs=[
            pl.BlockSpec((BM, H), lambda i: (i, 0)),
            pl.BlockSpec((BM, H), lambda i: (i, 0)),
            pl.BlockSpec((NW, BM), lambda i: (0, i)),
            pl.BlockSpec((1, H), lambda i: (0, 0)),
            pl.BlockSpec((BM, 1), lambda i: (i, 0)),
            pl.BlockSpec((2 * H, H), lambda i: (0, 0)),
            pl.BlockSpec((1, H), lambda i: (0, 0)),
        ],
        out_specs=pl.BlockSpec((G, H), lambda i: (0, 0)),
        out_shape=jax.ShapeDtypeStruct((G, H), _f32),
        scratch_shapes=[
            pltpu.VMEM((GP, H), _f32),
            pltpu.VMEM((GP, H), _f32),
            pltpu.VMEM((GP, H), _f32),
        ],
    )(usum[0], usum[1], sparts, b, bat2d, wfp, bfp)


def kernel(x, edge_index, deltaPhi, deltaEta, batch,
           W1, a_s1, a_d1, b1, W2, a_s2, a_d2, b2, Wf, bf):
    del deltaPhi, deltaEta

    e_total = edge_index.shape[1] + N
    tb = -(-e_total // (NS * EB))
    epb0 = 2 * max(1, round(tb * SLOW_FRAC / 2))
    epb1 = tb - epb0
    if epb1 % 2:
        epb1 += 1
    blocks = NS * (epb0 + epb1) + 2
    ep = -(-blocks * EB // (NW * 2 * EB)) * (NW * 2 * EB)
    ept = ep // NW

    loop = jnp.arange(N, dtype=_i32)
    pad_e = N + (jnp.arange(ep - e_total, dtype=_i32) % (NP - N))
    src = jnp.concatenate([edge_index[0].astype(_i32), loop, pad_e])
    dst = jnp.concatenate([edge_index[1].astype(_i32), loop, pad_e])

    x_pad = jnp.pad(x, ((0, NP - N), (0, 0)))
    bat2d = jnp.pad(batch.astype(_i32), (0, NP - N),
                    constant_values=G).reshape(NP, 1)

    def pack_aa(a_s, a_d):
        aa = jnp.zeros((H, 16), _f32)
        aa = aa.at[:, 0].set(a_s)
        aa = aa.at[:, 1].set(a_d)
        return aa

    h1, a1 = _tc_embed(x_pad, W1, pack_aa(a_s1, a_d1))
    ex1, sacc1 = _sc_scalar_pass(a1[:, 0], a1[:, 1], src, dst, ept, ep)
    usum1 = _sc_heavy_pass(h1, src, dst, ex1, epb0, epb1)

    h2, a2 = _tc_mid(usum1, sacc1, b1.reshape(1, H), W2, pack_aa(a_s2, a_d2))
    ex2, sacc2 = _sc_scalar_pass(a2[:, 0], a2[:, 1], src, dst, ept, ep)
    usum2 = _sc_heavy_pass(h2, src, dst, ex2, epb0, epb1)

    wfp = jnp.pad(Wf, ((0, 0), (0, H - OUT)))
    bfp = jnp.pad(bf, (0, H - OUT)).reshape(1, H)
    out = _tc_pool(usum2, sacc2, b2.reshape(1, H), bat2d, wfp, bfp)
    return out[:, :OUT]


# speedup vs baseline: 1.0004x; 1.0004x over previous
import functools

import jax
import jax.numpy as jnp
from jax import lax
from jax.experimental import pallas as pl
from jax.experimental.pallas import tpu as pltpu
from jax.experimental.pallas import tpu_sc as plsc

N = 10000
D = 128
H = 128
G = 64
OUT = 3

NP = 10240
BM = 512
NB = NP // BM
NC = 2
NS = 16
NW = NC * NS
EB = 112
GP = 72
RPT = NP // NS
NT = NP
ZC = 64
SLOW_FRAC = 0.50

_f32 = jnp.float32
_i32 = jnp.int32


def _sc_scalar_kernel(ept,
                      asrc_hbm, adst_hbm, src_hbm, dst_hbm,
                      ex_out, s_out,
                      asrc_v, adst_v, s_acc, src_v, dst_v, ex_v):
    cid = lax.axis_index("c")
    sid = lax.axis_index("s")
    wid = cid * NS + sid

    zero16 = jnp.zeros((16,), _f32)

    def _zs(j, _):
        s_acc[pl.ds(j * 16, 16)] = zero16
        return 0

    lax.fori_loop(0, NT // 16, _zs, 0)

    pltpu.sync_copy(asrc_hbm.at[pl.ds(0, NT)], asrc_v)
    pltpu.sync_copy(adst_hbm.at[pl.ds(0, NT)], adst_v)
    pltpu.sync_copy(src_hbm.at[pl.ds(wid * ept, ept)], src_v)
    pltpu.sync_copy(dst_hbm.at[pl.ds(wid * ept, ept)], dst_v)

    iota16 = lax.iota(_i32, 16)

    def _chunk(c, _):
        s16 = src_v[pl.ds(c * 16, 16)]
        d16 = dst_v[pl.ds(c * 16, 16)]
        av = plsc.load_gather(asrc_v, [s16])
        dv = plsc.load_gather(adst_v, [d16])
        e = av + dv
        e = jnp.where(e >= 0.0, e, 0.2 * e)
        ex = jnp.exp(e)
        ex_v[pl.ds(c * 16, 16)] = ex
        for l in range(16):
            plsc.addupdate_scatter(s_acc, [d16], ex, mask=iota16 == l)
        return 0

    lax.fori_loop(0, ept // 16, _chunk, 0)

    pltpu.sync_copy(ex_v, ex_out.at[pl.ds(wid * ept, ept)])
    pltpu.sync_copy(s_acc, s_out.at[wid, 0, pl.ds(0, NT)])


def _sc_scalar_pass(asrc, adst, src, dst, ept, ep):
    body = functools.partial(_sc_scalar_kernel, ept)
    ex, sparts = pl.kernel(
        body,
        out_type=[
            jax.ShapeDtypeStruct((ep,), _f32),
            jax.ShapeDtypeStruct((NW, 1, NP), _f32),
        ],
        mesh=plsc.VectorSubcoreMesh(core_axis_name="c", subcore_axis_name="s",
                                    num_cores=NC, num_subcores=NS),
        compiler_params=pltpu.CompilerParams(needs_layout_passes=False),
        scratch_types=[
            pltpu.VMEM((NT,), _f32),
            pltpu.VMEM((NT,), _f32),
            pltpu.VMEM((NT,), _f32),
            pltpu.VMEM((ept,), _i32),
            pltpu.VMEM((ept,), _i32),
            pltpu.VMEM((ept,), _f32),
        ],
    )(asrc, adst, src, dst)
    return ex, sparts.reshape(NW, NP)


def _sc_heavy_kernel(epb0, epb1,
                     h_hbm, src_hbm, dst_hbm, ex_hbm,
                     usum_out,
                     usum_sh,
                     src0, src1, dst0, dst1, exb0, exb1, dsc0, dsc1,
                     rows0, rows1, sb0, sb1,
                     isem0, isem1, gsem0, gsem1, ssem0, ssem1):
    cid = lax.axis_index("c")
    sid = lax.axis_index("s")
    wid = cid * NS + sid

    srcs = (src0, src1)
    dsts = (dst0, dst1)
    exbs = (exb0, exb1)
    dscs = (dsc0, dsc1)
    rows = (rows0, rows1)
    sbs = (sb0, sb1)
    isems = (isem0, isem1)
    gsems = (gsem0, gsem1)
    ssems = (ssem0, ssem1)

    zero16 = jnp.zeros((16,), _f32)

    def _zrow(j, _):
        for k in range(8):
            sb0[j, pl.ds(k * 16, 16)] = zero16
        return 0

    lax.fori_loop(0, EB, _zrow, 0)

    for i in range(RPT // ZC):
        base = sid * RPT + i * ZC
        pltpu.sync_copy(sb0.at[pl.ds(0, ZC), :],
                        usum_sh.at[pl.ds(base, ZC), :])

    plsc.subcore_barrier()

    epbc = jnp.where(cid == 0, epb0, epb1)
    ebase = jnp.where(cid == 0, sid * epb0, NS * epb0 + sid * epb1) * EB

    def _idx_start(blk, p):
        off = ebase + blk * EB
        pltpu.make_async_copy(src_hbm.at[pl.ds(off, EB)], srcs[p],
                              isems[p]).start()
        pltpu.make_async_copy(dst_hbm.at[pl.ds(off, EB)], dsts[p],
                              isems[p]).start()
        pltpu.make_async_copy(ex_hbm.at[pl.ds(off, EB)], exbs[p],
                              isems[p]).start()

    def _idx_wait(blk, p):
        off = ebase + blk * EB
        pltpu.make_async_copy(src_hbm.at[pl.ds(off, EB)], srcs[p],
                              isems[p]).wait()
        pltpu.make_async_copy(dst_hbm.at[pl.ds(off, EB)], dsts[p],
                              isems[p]).wait()
        pltpu.make_async_copy(ex_hbm.at[pl.ds(off, EB)], exbs[p],
                              isems[p]).wait()

    def _gather_start(p):
        pltpu.make_async_copy(h_hbm.at[srcs[p]], rows[p], gsems[p]).start()

    def _gather_wait(p):
        pltpu.make_async_copy(h_hbm.at[srcs[p]], rows[p], gsems[p]).wait()

    def _scat_start(p):
        pltpu.async_copy(sbs[p], usum_sh.at[dscs[p]], ssems[p], add=True)

    def _scat_wait(p):
        pltpu.make_async_copy(sbs[p], usum_sh.at[dscs[p]], ssems[p]).wait()

    _idx_start(0, 0)
    _idx_start(1, 1)
    _idx_wait(0, 0)
    _gather_start(0)

    def _step(blk, p, first=False):
        _idx_wait(blk + 1, 1 - p)
        _gather_start(1 - p)
        _gather_wait(p)

        dst_v, ex_v, dsc_v, rows_v, sb_v = (dsts[p], exbs[p], dscs[p],
                                            rows[p], sbs[p])
        for k in range(EB // 16):
            dsc_v[pl.ds(k * 16, 16)] = dst_v[pl.ds(k * 16, 16)]

        if not first:
            _scat_wait(p)

        def _scale(j, _):
            b = plsc.load_gather(ex_v, [jnp.full((16,), j, _i32)])
            for k in range(4):
                v = plsc.bitcast(rows_v[j, pl.ds(k * 16, 16)], jnp.uint32)
                lo = plsc.bitcast(v << 16, _f32)
                hi = plsc.bitcast(v & jnp.uint32(0xFFFF0000), _f32)
                sb_v[j, pl.ds(k * 16, 16)] = lo * b
                sb_v[j, pl.ds(64 + k * 16, 16)] = hi * b
            return 0

        lax.fori_loop(0, EB, _scale, 0)

        _scat_start(p)
        _idx_start(blk + 2, p)

    _step(0, 0, first=True)
    _step(1, 1, first=True)

    def _pair(o, _):
        _step(2 * o + 2, 0)
        _step(2 * o + 3, 1)
        return 0

    lax.fori_loop(0, (epbc - 2) // 2, _pair, 0)

    _idx_wait(epbc + 1, 1)
    _gather_wait(0)
    _scat_wait(0)
    _scat_wait(1)

    plsc.subcore_barrier()

    pltpu.sync_copy(usum_sh.at[pl.ds(sid * RPT, RPT), :],
                    usum_out.at[cid, pl.ds(sid * RPT, RPT), :])


def _sc_heavy_pass(h, src, dst, ex, epb0, epb1):
    body = functools.partial(_sc_heavy_kernel, epb0, epb1)
    return pl.kernel(
        body,
        out_type=jax.ShapeDtypeStruct((NC, NP, H), _f32),
        mesh=plsc.VectorSubcoreMesh(core_axis_name="c", subcore_axis_name="s",
                                    num_cores=NC, num_subcores=NS),
        compiler_params=pltpu.CompilerParams(needs_layout_passes=False,
                                             use_tc_tiling_on_sc=False),
        scratch_types=[
            pltpu.VMEM_SHARED((NP, H), _f32),
            pltpu.VMEM((EB,), _i32),
            pltpu.VMEM((EB,), _i32),
            pltpu.VMEM((EB,), _i32),
            pltpu.VMEM((EB,), _i32),
            pltpu.VMEM((EB,), _f32),
            pltpu.VMEM((EB,), _f32),
            pltpu.VMEM((EB,), _i32),
            pltpu.VMEM((EB,), _i32),
            pltpu.VMEM((EB, H // 2), _f32),
            pltpu.VMEM((EB, H // 2), _f32),
            pltpu.VMEM((EB, H), _f32),
            pltpu.VMEM((EB, H), _f32),
            pltpu.SemaphoreType.DMA,
            pltpu.SemaphoreType.DMA,
            pltpu.SemaphoreType.DMA,
            pltpu.SemaphoreType.DMA,
            pltpu.SemaphoreType.DMA,
            pltpu.SemaphoreType.DMA,
        ],
    )(h, src, dst, ex)


def _pack_rows(hv):
    lob = lax.bitcast_convert_type(hv[:, :64].astype(jnp.bfloat16),
                                   jnp.uint16).astype(jnp.uint32)
    hib = lax.bitcast_convert_type(hv[:, 64:].astype(jnp.bfloat16),
                                   jnp.uint16).astype(jnp.uint32)
    return lax.bitcast_convert_type(lob | (hib << 16), _f32)


def _tc_embed_body(x_ref, w_ref, aa_ref, h_ref, a_ref):
    xr = jnp.maximum(x_ref[...], 0.0)
    hv = jnp.dot(xr, w_ref[...], preferred_element_type=_f32)
    h_ref[...] = _pack_rows(hv)
    a_ref[...] = jnp.dot(hv, aa_ref[...], preferred_element_type=_f32)


def _tc_embed(x_pad, w, aa):
    return pl.pallas_call(
        _tc_embed_body,
        grid=(NB,),
        in_specs=[
            pl.BlockSpec((BM, D), lambda i: (i, 0)),
            pl.BlockSpec((D, H), lambda i: (0, 0)),
            pl.BlockSpec((H, 16), lambda i: (0, 0)),
        ],
        out_specs=[
            pl.BlockSpec((BM, H // 2), lambda i: (i, 0)),
            pl.BlockSpec((BM, 16), lambda i: (i, 0)),
        ],
        out_shape=[
            jax.ShapeDtypeStruct((NP, H // 2), _f32),
            jax.ShapeDtypeStruct((NP, 16), _f32),
        ],
    )(x_pad, w, aa)


def _den_from_parts(sp):
    ones = jnp.ones((NW, 8), _f32)
    tot = lax.dot_general(sp, ones, (((0,), (0,)), ((), ())),
                          preferred_element_type=_f32)
    return jnp.maximum(tot[:, 0:1], 1e-16)


def _tc_mid_body(u0_ref, u1_ref, sp_ref, b_ref, w_ref, aa_ref,
                 h_ref, a_ref):
    i = pl.program_id(0)
    u = u0_ref[...] + u1_ref[...]
    den = _den_from_parts(sp_ref[...])
    h1 = u / den + b_ref[...]
    x2 = jnp.maximum(h1, 0.0)
    ridx = lax.broadcasted_iota(_i32, (BM, 1), 0) + i * BM
    x2 = jnp.where(ridx < N, x2, 0.0)
    hv = jnp.dot(x2, w_ref[...], preferred_element_type=_f32)
    h_ref[...] = _pack_rows(hv)
    a_ref[...] = jnp.dot(hv, aa_ref[...], preferred_element_type=_f32)


def _tc_mid(usum, sparts, b, w, aa):
    return pl.pallas_call(
        _tc_mid_body,
        grid=(NB,),
        in_specs=[
            pl.BlockSpec((BM, H), lambda i: (i, 0)),
            pl.BlockSpec((BM, H), lambda i: (i, 0)),
            pl.BlockSpec((NW, BM), lambda i: (0, i)),
            pl.BlockSpec((1, H), lambda i: (0, 0)),
            pl.BlockSpec((H, H), lambda i: (0, 0)),
            pl.BlockSpec((H, 16), lambda i: (0, 0)),
        ],
        out_specs=[
            pl.BlockSpec((BM, H // 2), lambda i: (i, 0)),
            pl.BlockSpec((BM, 16), lambda i: (i, 0)),
        ],
        out_shape=[
            jax.ShapeDtypeStruct((NP, H // 2), _f32),
            jax.ShapeDtypeStruct((NP, 16), _f32),
        ],
    )(usum[0], usum[1], sparts, b, w, aa)


def _tc_pool_body(u0_ref, u1_ref, sp_ref, b_ref, bat_ref,
                  wf_ref, bf_ref, out_ref, gsum, gmax, gcnt):
    i = pl.program_id(0)

    @pl.when(i == 0)
    def _():
        gsum[...] = jnp.zeros((GP, H), _f32)
        gcnt[...] = jnp.zeros((GP, H), _f32)
        gmax[...] = jnp.full((GP, H), -jnp.inf, _f32)

    u = u0_ref[...] + u1_ref[...]
    den = _den_from_parts(sp_ref[...])
    h2 = u / den + b_ref[...]
    ridx = lax.broadcasted_iota(_i32, (BM, 1), 0) + i * BM
    h2 = jnp.where(ridx < N, h2, 0.0)

    bcol = bat_ref[...]
    giota = lax.broadcasted_iota(_i32, (BM, GP), 1)
    oh = (bcol == giota).astype(_f32)
    dn = (((0,), (0,)), ((), ()))
    gsum[...] = gsum[...] + lax.dot_general(oh, h2, dn,
                                            preferred_element_type=_f32)
    gcnt[...] = gcnt[...] + lax.dot_general(oh, jnp.ones_like(h2), dn,
                                            preferred_element_type=_f32)

    b_lo = jnp.min(bcol)
    b_hi = jnp.max(bcol)

    def _gmax(g, _):
        @pl.when(jnp.logical_and(g >= b_lo, g <= b_hi))
        def _():
            m = bcol == g
            mx = jnp.max(jnp.where(m, h2, -jnp.inf), axis=0, keepdims=True)
            gmax[pl.ds(g, 1), :] = jnp.maximum(gmax[pl.ds(g, 1), :], mx)
        return 0

    lax.fori_loop(0, G, _gmax, 0)

    @pl.when(i == NB - 1)
    def _():
        cnt = gcnt[...]
        gmx = jnp.where(cnt > 0.0, gmax[...], 0.0)
        gmn = gsum[...] / jnp.maximum(cnt, 1.0)
        pooled = jnp.concatenate([gmx[:G], gmn[:G]], axis=1)
        out_ref[...] = (jnp.dot(pooled, wf_ref[...],
                                preferred_element_type=_f32) + bf_ref[...])


def _tc_pool(usum, sparts, b, bat2d, wfp, bfp):
    return pl.pallas_call(
        _tc_pool_body,
        grid=(NB,),
        in_specs=[
            pl.BlockSpec((BM, H), lambda i: (i, 0)),
            pl.BlockSpec((BM, H), lambda i: (i, 0)),
            pl.BlockSpec((NW, BM), lambda i: (0, i)),
            pl.BlockSpec((1, H), lambda i: (0, 0)),
            pl.BlockSpec((BM, 1), lambda i: (i, 0)),
            pl.BlockSpec((2 * H, H), lambda i: (0, 0)),
            pl.BlockSpec((1, H), lambda i: (0, 0)),
        ],
        out_specs=pl.BlockSpec((G, H), lambda i: (0, 0)),
        out_shape=jax.ShapeDtypeStruct((G, H), _f32),
        scratch_shapes=[
            pltpu.VMEM((GP, H), _f32),
            pltpu.VMEM((GP, H), _f32),
            pltpu.VMEM((GP, H), _f32),
        ],
    )(usum[0], usum[1], sparts, b, bat2d, wfp, bfp)


def kernel(x, edge_index, deltaPhi, deltaEta, batch,
           W1, a_s1, a_d1, b1, W2, a_s2, a_d2, b2, Wf, bf):
    del deltaPhi, deltaEta

    e_total = edge_index.shape[1] + N
    tb = -(-e_total // (NS * EB))
    epb0 = 2 * max(1, round(tb * SLOW_FRAC / 2))
    epb1 = tb - epb0
    if epb1 % 2:
        epb1 += 1
    blocks = NS * (epb0 + epb1) + 2
    ep = -(-blocks * EB // (NW * 2 * EB)) * (NW * 2 * EB)
    ept = ep // NW

    loop = jnp.arange(N, dtype=_i32)
    pad_e = N + (jnp.arange(ep - e_total, dtype=_i32) % (NP - N))
    src = jnp.concatenate([edge_index[0].astype(_i32), loop, pad_e])
    dst = jnp.concatenate([edge_index[1].astype(_i32), loop, pad_e])

    x_pad = jnp.pad(x, ((0, NP - N), (0, 0)))
    bat2d = jnp.pad(batch.astype(_i32), (0, NP - N),
                    constant_values=G).reshape(NP, 1)

    def pack_aa(a_s, a_d):
        aa = jnp.zeros((H, 16), _f32)
        aa = aa.at[:, 0].set(a_s)
        aa = aa.at[:, 1].set(a_d)
        return aa

    h1, a1 = _tc_embed(x_pad, W1, pack_aa(a_s1, a_d1))
    ex1, sacc1 = _sc_scalar_pass(a1[:, 0], a1[:, 1], src, dst, ept, ep)
    usum1 = _sc_heavy_pass(h1, src, dst, ex1, epb0, epb1)

    h2, a2 = _tc_mid(usum1, sacc1, b1.reshape(1, H), W2, pack_aa(a_s2, a_d2))
    ex2, sacc2 = _sc_scalar_pass(a2[:, 0], a2[:, 1], src, dst, ept, ep)
    usum2 = _sc_heavy_pass(h2, src, dst, ex2, epb0, epb1)

    wfp = jnp.pad(Wf, ((0, 0), (0, H - OUT)))
    bfp = jnp.pad(bf, (0, H - OUT)).reshape(1, H)
    out = _tc_pool(usum2, sacc2, b2.reshape(1, H), bat2d, wfp, bfp)
    return out[:, :OUT]


# speedup vs baseline: 1.4484x; 1.4479x over previous
import functools

import jax
import jax.numpy as jnp
from jax import lax
from jax.experimental import pallas as pl
from jax.experimental.pallas import tpu as pltpu
from jax.experimental.pallas import tpu_sc as plsc

N = 10000
D = 128
H = 128
G = 64
OUT = 3

NP = 10240
BM = 512
NB = NP // BM
NC = 2
NS = 16
NW = NC * NS
EB = 128
GP = 72
RPT = NP // NS
NT = NP
ZC = 64
SLOW_FRAC = 0.50

_f32 = jnp.float32
_i32 = jnp.int32


def _sc_scalar_kernel(ept,
                      asrc_hbm, adst_hbm, src_hbm, dst_hbm,
                      ex_out, s_out,
                      asrc_v, adst_v, s_acc, src_v, dst_v, ex_v):
    cid = lax.axis_index("c")
    sid = lax.axis_index("s")
    wid = cid * NS + sid

    zero16 = jnp.zeros((16,), _f32)

    def _zs(j, _):
        s_acc[pl.ds(j * 16, 16)] = zero16
        return 0

    lax.fori_loop(0, NT // 16, _zs, 0)

    pltpu.sync_copy(asrc_hbm.at[pl.ds(0, NT)], asrc_v)
    pltpu.sync_copy(adst_hbm.at[pl.ds(0, NT)], adst_v)
    pltpu.sync_copy(src_hbm.at[pl.ds(wid * ept, ept)], src_v)
    pltpu.sync_copy(dst_hbm.at[pl.ds(wid * ept, ept)], dst_v)

    iota16 = lax.iota(_i32, 16)

    def _chunk(c, _):
        s16 = src_v[pl.ds(c * 16, 16)]
        d16 = dst_v[pl.ds(c * 16, 16)]
        av = plsc.load_gather(asrc_v, [s16])
        dv = plsc.load_gather(adst_v, [d16])
        e = av + dv
        e = jnp.where(e >= 0.0, e, 0.2 * e)
        ex = jnp.exp(e)
        ex_v[pl.ds(c * 16, 16)] = ex
        for l in range(16):
            plsc.addupdate_scatter(s_acc, [d16], ex, mask=iota16 == l)
        return 0

    lax.fori_loop(0, ept // 16, _chunk, 0)

    pltpu.sync_copy(ex_v, ex_out.at[pl.ds(wid * ept, ept)])
    pltpu.sync_copy(s_acc, s_out.at[wid, 0, pl.ds(0, NT)])


def _sc_scalar_pass(asrc, adst, src, dst, ept, ep):
    body = functools.partial(_sc_scalar_kernel, ept)
    ex, sparts = pl.kernel(
        body,
        out_type=[
            jax.ShapeDtypeStruct((ep,), _f32),
            jax.ShapeDtypeStruct((NW, 1, NP), _f32),
        ],
        mesh=plsc.VectorSubcoreMesh(core_axis_name="c", subcore_axis_name="s",
                                    num_cores=NC, num_subcores=NS),
        compiler_params=pltpu.CompilerParams(needs_layout_passes=False),
        scratch_types=[
            pltpu.VMEM((NT,), _f32),
            pltpu.VMEM((NT,), _f32),
            pltpu.VMEM((NT,), _f32),
            pltpu.VMEM((ept,), _i32),
            pltpu.VMEM((ept,), _i32),
            pltpu.VMEM((ept,), _f32),
        ],
    )(asrc, adst, src, dst)
    return ex, sparts.reshape(NW, NP)


def _sc_heavy_kernel(epb0, epb1,
                     h_hbm, src_hbm, dst_hbm, ex_hbm,
                     usum_out,
                     usum_sh,
                     src0, src1, dst0, dst1, exb0, exb1, dsc0, dsc1,
                     rows0, rows1,
                     isem0, isem1, gsem0, gsem1, ssem0, ssem1):
    cid = lax.axis_index("c")
    sid = lax.axis_index("s")
    wid = cid * NS + sid

    srcs = (src0, src1)
    dsts = (dst0, dst1)
    exbs = (exb0, exb1)
    dscs = (dsc0, dsc1)
    rows = (rows0, rows1)
    isems = (isem0, isem1)
    gsems = (gsem0, gsem1)
    ssems = (ssem0, ssem1)

    zero16 = jnp.zeros((16,), _f32)

    def _zrow(j, _):
        for k in range(8):
            rows0[j, pl.ds(k * 16, 16)] = zero16
        return 0

    lax.fori_loop(0, EB, _zrow, 0)

    for i in range(RPT // ZC):
        base = sid * RPT + i * ZC
        pltpu.sync_copy(rows0.at[pl.ds(0, ZC), :],
                        usum_sh.at[pl.ds(base, ZC), :])

    plsc.subcore_barrier()

    epbc = jnp.where(cid == 0, epb0, epb1)
    ebase = jnp.where(cid == 0, sid * epb0, NS * epb0 + sid * epb1) * EB

    def _idx_start(blk, p):
        off = ebase + blk * EB
        pltpu.make_async_copy(src_hbm.at[pl.ds(off, EB)], srcs[p],
                              isems[p]).start()
        pltpu.make_async_copy(dst_hbm.at[pl.ds(off, EB)], dsts[p],
                              isems[p]).start()
        pltpu.make_async_copy(ex_hbm.at[pl.ds(off, EB)], exbs[p],
                              isems[p]).start()

    def _idx_wait(blk, p):
        off = ebase + blk * EB
        pltpu.make_async_copy(src_hbm.at[pl.ds(off, EB)], srcs[p],
                              isems[p]).wait()
        pltpu.make_async_copy(dst_hbm.at[pl.ds(off, EB)], dsts[p],
                              isems[p]).wait()
        pltpu.make_async_copy(ex_hbm.at[pl.ds(off, EB)], exbs[p],
                              isems[p]).wait()

    def _gather_start(p):
        pltpu.make_async_copy(h_hbm.at[srcs[p]], rows[p], gsems[p]).start()

    def _gather_wait(p):
        pltpu.make_async_copy(h_hbm.at[srcs[p]], rows[p], gsems[p]).wait()

    def _scat_start(p):
        pltpu.async_copy(rows[p], usum_sh.at[dscs[p]], ssems[p], add=True)

    def _scat_wait(p):
        pltpu.make_async_copy(rows[p], usum_sh.at[dscs[p]], ssems[p]).wait()

    _idx_start(0, 0)
    _idx_start(1, 1)
    _idx_wait(0, 0)
    _gather_start(0)

    def _step(blk, p, first=False):
        _idx_wait(blk + 1, 1 - p)
        if not first:
            _scat_wait(1 - p)
        _gather_start(1 - p)
        _gather_wait(p)

        dst_v, ex_v, dsc_v, rows_v = dsts[p], exbs[p], dscs[p], rows[p]
        for k in range(8):
            dsc_v[pl.ds(k * 16, 16)] = dst_v[pl.ds(k * 16, 16)]

        def _scale(j, _):
            b = plsc.load_gather(ex_v, [jnp.full((16,), j, _i32)])
            for k in range(8):
                rows_v[j, pl.ds(k * 16, 16)] = rows_v[j, pl.ds(k * 16, 16)] * b
            return 0

        lax.fori_loop(0, EB, _scale, 0)

        _scat_start(p)
        _idx_start(blk + 2, p)

    _step(0, 0, first=True)

    def _pair(o, _):
        _step(2 * o + 1, 1)
        _step(2 * o + 2, 0)
        return 0

    lax.fori_loop(0, (epbc - 2) // 2, _pair, 0)
    _step(epbc - 1, 1)

    _idx_wait(epbc + 1, 1)
    _gather_wait(0)
    _scat_wait(1)

    plsc.subcore_barrier()

    pltpu.sync_copy(usum_sh.at[pl.ds(sid * RPT, RPT), :],
                    usum_out.at[cid, pl.ds(sid * RPT, RPT), :])


def _sc_heavy_pass(h, src, dst, ex, epb0, epb1):
    body = functools.partial(_sc_heavy_kernel, epb0, epb1)
    return pl.kernel(
        body,
        out_type=jax.ShapeDtypeStruct((NC, NP, H), _f32),
        mesh=plsc.VectorSubcoreMesh(core_axis_name="c", subcore_axis_name="s",
                                    num_cores=NC, num_subcores=NS),
        compiler_params=pltpu.CompilerParams(needs_layout_passes=False),
        scratch_types=[
            pltpu.VMEM_SHARED((NP, H), _f32),
            pltpu.VMEM((EB,), _i32),
            pltpu.VMEM((EB,), _i32),
            pltpu.VMEM((EB,), _i32),
            pltpu.VMEM((EB,), _i32),
            pltpu.VMEM((EB,), _f32),
            pltpu.VMEM((EB,), _f32),
            pltpu.VMEM((EB,), _i32),
            pltpu.VMEM((EB,), _i32),
            pltpu.VMEM((EB, H), _f32),
            pltpu.VMEM((EB, H), _f32),
            pltpu.SemaphoreType.DMA,
            pltpu.SemaphoreType.DMA,
            pltpu.SemaphoreType.DMA,
            pltpu.SemaphoreType.DMA,
            pltpu.SemaphoreType.DMA,
            pltpu.SemaphoreType.DMA,
        ],
    )(h, src, dst, ex)


def _tc_embed_body(x_ref, w_ref, aa_ref, h_ref, a_ref):
    xr = jnp.maximum(x_ref[...], 0.0)
    hv = jnp.dot(xr, w_ref[...], preferred_element_type=_f32)
    h_ref[...] = hv
    a_ref[...] = jnp.dot(hv, aa_ref[...], preferred_element_type=_f32)


def _tc_embed(x_pad, w, aa):
    return pl.pallas_call(
        _tc_embed_body,
        grid=(NB,),
        in_specs=[
            pl.BlockSpec((BM, D), lambda i: (i, 0)),
            pl.BlockSpec((D, H), lambda i: (0, 0)),
            pl.BlockSpec((H, 16), lambda i: (0, 0)),
        ],
        out_specs=[
            pl.BlockSpec((BM, H), lambda i: (i, 0)),
            pl.BlockSpec((BM, 16), lambda i: (i, 0)),
        ],
        out_shape=[
            jax.ShapeDtypeStruct((NP, H), _f32),
            jax.ShapeDtypeStruct((NP, 16), _f32),
        ],
    )(x_pad, w, aa)


def _den_from_parts(sp):
    ones = jnp.ones((NW, 8), _f32)
    tot = lax.dot_general(sp, ones, (((0,), (0,)), ((), ())),
                          preferred_element_type=_f32)
    return jnp.maximum(tot[:, 0:1], 1e-16)


def _tc_mid_body(u0_ref, u1_ref, sp_ref, b_ref, w_ref, aa_ref,
                 h_ref, a_ref):
    i = pl.program_id(0)
    u = u0_ref[...] + u1_ref[...]
    den = _den_from_parts(sp_ref[...])
    h1 = u / den + b_ref[...]
    x2 = jnp.maximum(h1, 0.0)
    ridx = lax.broadcasted_iota(_i32, (BM, 1), 0) + i * BM
    x2 = jnp.where(ridx < N, x2, 0.0)
    hv = jnp.dot(x2, w_ref[...], preferred_element_type=_f32)
    h_ref[...] = hv
    a_ref[...] = jnp.dot(hv, aa_ref[...], preferred_element_type=_f32)


def _tc_mid(usum, sparts, b, w, aa):
    return pl.pallas_call(
        _tc_mid_body,
        grid=(NB,),
        in_specs=[
            pl.BlockSpec((BM, H), lambda i: (i, 0)),
            pl.BlockSpec((BM, H), lambda i: (i, 0)),
            pl.BlockSpec((NW, BM), lambda i: (0, i)),
            pl.BlockSpec((1, H), lambda i: (0, 0)),
            pl.BlockSpec((H, H), lambda i: (0, 0)),
            pl.BlockSpec((H, 16), lambda i: (0, 0)),
        ],
        out_specs=[
            pl.BlockSpec((BM, H), lambda i: (i, 0)),
            pl.BlockSpec((BM, 16), lambda i: (i, 0)),
        ],
        out_shape=[
            jax.ShapeDtypeStruct((NP, H), _f32),
            jax.ShapeDtypeStruct((NP, 16), _f32),
        ],
    )(usum[0], usum[1], sparts, b, w, aa)


def _tc_pool_body(u0_ref, u1_ref, sp_ref, b_ref, bat_ref,
                  wf_ref, bf_ref, out_ref, gsum, gmax, gcnt):
    i = pl.program_id(0)

    @pl.when(i == 0)
    def _():
        gsum[...] = jnp.zeros((GP, H), _f32)
        gcnt[...] = jnp.zeros((GP, H), _f32)
        gmax[...] = jnp.full((GP, H), -jnp.inf, _f32)

    u = u0_ref[...] + u1_ref[...]
    den = _den_from_parts(sp_ref[...])
    h2 = u / den + b_ref[...]
    ridx = lax.broadcasted_iota(_i32, (BM, 1), 0) + i * BM
    h2 = jnp.where(ridx < N, h2, 0.0)

    bcol = bat_ref[...]
    giota = lax.broadcasted_iota(_i32, (BM, GP), 1)
    oh = (bcol == giota).astype(_f32)
    dn = (((0,), (0,)), ((), ()))
    gsum[...] = gsum[...] + lax.dot_general(oh, h2, dn,
                                            preferred_element_type=_f32)
    gcnt[...] = gcnt[...] + lax.dot_general(oh, jnp.ones_like(h2), dn,
                                            preferred_element_type=_f32)

    b_lo = jnp.min(bcol)
    b_hi = jnp.max(bcol)

    def _gmax(g, _):
        @pl.when(jnp.logical_and(g >= b_lo, g <= b_hi))
        def _():
            m = bcol == g
            mx = jnp.max(jnp.where(m, h2, -jnp.inf), axis=0, keepdims=True)
            gmax[pl.ds(g, 1), :] = jnp.maximum(gmax[pl.ds(g, 1), :], mx)
        return 0

    lax.fori_loop(0, G, _gmax, 0)

    @pl.when(i == NB - 1)
    def _():
        cnt = gcnt[...]
        gmx = jnp.where(cnt > 0.0, gmax[...], 0.0)
        gmn = gsum[...] / jnp.maximum(cnt, 1.0)
        pooled = jnp.concatenate([gmx[:G], gmn[:G]], axis=1)
        out_ref[...] = (jnp.dot(pooled, wf_ref[...],
                                preferred_element_type=_f32) + bf_ref[...])


def _tc_pool(usum, sparts, b, bat2d, wfp, bfp):
    return pl.pallas_call(
        _tc_pool_body,
        grid=(NB,),
        in_specs=[
            pl.BlockSpec((BM, H), lambda i: (i, 0)),
            pl.BlockSpec((BM, H), lambda i: (i, 0)),
            pl.BlockSpec((NW, BM), lambda i: (0, i)),
            pl.BlockSpec((1, H), lambda i: (0, 0)),
            pl.BlockSpec((BM, 1), lambda i: (i, 0)),
            pl.BlockSpec((2 * H, H), lambda i: (0, 0)),
            pl.BlockSpec((1, H), lambda i: (0, 0)),
        ],
        out_specs=pl.BlockSpec((G, H), lambda i: (0, 0)),
        out_shape=jax.ShapeDtypeStruct((G, H), _f32),
        scratch_shapes=[
            pltpu.VMEM((GP, H), _f32),
            pltpu.VMEM((GP, H), _f32),
            pltpu.VMEM((GP, H), _f32),
        ],
    )(usum[0], usum[1], sparts, b, bat2d, wfp, bfp)


def kernel(x, edge_index, deltaPhi, deltaEta, batch,
           W1, a_s1, a_d1, b1, W2, a_s2, a_d2, b2, Wf, bf):
    del deltaPhi, deltaEta

    e_total = edge_index.shape[1] + N
    tb = -(-e_total // (NS * EB))
    epb0 = 2 * max(1, round(tb * SLOW_FRAC / 2))
    epb1 = tb - epb0
    if epb1 % 2:
        epb1 += 1
    blocks = NS * (epb0 + epb1) + 2
    ep = -(-blocks * EB // (NW * 2 * EB)) * (NW * 2 * EB)
    ept = ep // NW

    loop = jnp.arange(N, dtype=_i32)
    pad_e = N + (jnp.arange(ep - e_total, dtype=_i32) % (NP - N))
    src = jnp.concatenate([edge_index[0].astype(_i32), loop, pad_e])
    dst = jnp.concatenate([edge_index[1].astype(_i32), loop, pad_e])

    x_pad = jnp.pad(x, ((0, NP - N), (0, 0)))
    bat2d = jnp.pad(batch.astype(_i32), (0, NP - N),
                    constant_values=G).reshape(NP, 1)

    def pack_aa(a_s, a_d):
        aa = jnp.zeros((H, 16), _f32)
        aa = aa.at[:, 0].set(a_s)
        aa = aa.at[:, 1].set(a_d)
        return aa

    h1, a1 = _tc_embed(x_pad, W1, pack_aa(a_s1, a_d1))
    ex1, sacc1 = _sc_scalar_pass(a1[:, 0], a1[:, 1], src, dst, ept, ep)
    usum1 = _sc_heavy_pass(h1, src, dst, ex1, epb0, epb1)

    h2, a2 = _tc_mid(usum1, sacc1, b1.reshape(1, H), W2, pack_aa(a_s2, a_d2))
    ex2, sacc2 = _sc_scalar_pass(a2[:, 0], a2[:, 1], src, dst, ept, ep)
    usum2 = _sc_heavy_pass(h2, src, dst, ex2, epb0, epb1)

    wfp = jnp.pad(Wf, ((0, 0), (0, H - OUT)))
    bfp = jnp.pad(bf, (0, H - OUT)).reshape(1, H)
    out = _tc_pool(usum2, sacc2, b2.reshape(1, H), bat2d, wfp, bfp)
    return out[:, :OUT]
